# Initial kernel scaffold; baseline (speedup 1.0000x reference)
import functools

import jax
import jax.numpy as jnp
from jax import lax
from jax.experimental import pallas as pl
from jax.experimental.pallas import tpu as pltpu
from jax.experimental.pallas import tpu_sc as plsc

N = 10000
E = 320000
D = 128
HOPS = 3
ALPHA = 0.5

NC = 2
NS = 16
NW = NC * NS
CH = 128
NCHUNK = 80
E_PAD = NW * NCHUNK * CH
N_PAD = 10240
RPS = N_PAD // NS
L = 16

_mesh = plsc.VectorSubcoreMesh(core_axis_name="c", subcore_axis_name="s")


@functools.partial(
    pl.kernel,
    out_type=jax.ShapeDtypeStruct((NC, N_PAD, L), jnp.float32),
    mesh=_mesh,
    scratch_types=[
        pltpu.VMEM_SHARED((N_PAD, L), jnp.float32),
        pltpu.VMEM((NCHUNK, CH), jnp.int32),
        pltpu.VMEM((NCHUNK, CH), jnp.float32),
        pltpu.VMEM((CH, L), jnp.float32),
    ],
)
def _denom_partial_kernel(dst_hbm, w_hbm, out_hbm, acc, dst_v, w_v, w16_v):
    cid = lax.axis_index("c")
    sid = lax.axis_index("s")
    wid = cid * NS + sid
    pltpu.sync_copy(dst_hbm.at[wid], dst_v)
    pltpu.sync_copy(w_hbm.at[wid], w_v)

    zero = jnp.zeros((L,), jnp.float32)

    @pl.loop(0, CH)
    def _(e):
        w16_v[e, :] = zero

    @pl.loop(0, RPS // CH)
    def _(b):
        pltpu.sync_copy(w16_v, acc.at[pl.ds(sid * RPS + b * CH, CH)])

    plsc.subcore_barrier()

    @pl.loop(0, NCHUNK)
    def _(j):
        @pl.loop(0, CH)
        def _(e):
            w16_v[e, :] = jnp.full((L,), w_v[j, e], jnp.float32)

        pltpu.sync_copy(w16_v, acc.at[dst_v.at[j]], add=True)

    plsc.subcore_barrier()
    pltpu.sync_copy(acc.at[pl.ds(sid * RPS, RPS)],
                    out_hbm.at[cid, pl.ds(sid * RPS, RPS)])


@functools.partial(
    pl.kernel,
    out_type=jax.ShapeDtypeStruct((NW, NCHUNK, CH, L), jnp.float32),
    mesh=_mesh,
    scratch_types=[
        pltpu.VMEM((NCHUNK, CH), jnp.int32),
        pltpu.VMEM((NCHUNK, CH), jnp.float32),
        pltpu.VMEM((CH, L), jnp.float32),
        pltpu.VMEM((CH, L), jnp.float32),
        pltpu.VMEM((CH, L), jnp.float32),
    ],
)
def _val_kernel(dst_hbm, w_hbm, p0_hbm, p1_hbm, out_hbm,
                dst_v, w_v, d0_v, d1_v, v_v):
    cid = lax.axis_index("c")
    sid = lax.axis_index("s")
    wid = cid * NS + sid
    pltpu.sync_copy(dst_hbm.at[wid], dst_v)
    pltpu.sync_copy(w_hbm.at[wid], w_v)

    eps = jnp.full((L,), 1e-12, jnp.float32)

    @pl.loop(0, NCHUNK)
    def _(j):
        pltpu.sync_copy(p0_hbm.at[dst_v.at[j]], d0_v)
        pltpu.sync_copy(p1_hbm.at[dst_v.at[j]], d1_v)

        @pl.loop(0, CH)
        def _(e):
            d = d0_v[e, :] + d1_v[e, :]
            v_v[e, :] = jnp.full((L,), w_v[j, e], jnp.float32) / jnp.maximum(d, eps)

        pltpu.sync_copy(v_v, out_hbm.at[wid, j])


@functools.partial(
    pl.kernel,
    out_type=jax.ShapeDtypeStruct((NC, N_PAD, D), jnp.float32),
    mesh=_mesh,
    scratch_types=[
        pltpu.VMEM_SHARED((N_PAD, D), jnp.float32),
        pltpu.VMEM((NCHUNK, CH), jnp.int32),
        pltpu.VMEM((NCHUNK, CH), jnp.int32),
        pltpu.VMEM((CH, D), jnp.float32),
        pltpu.VMEM((CH, D), jnp.float32),
        pltpu.VMEM((CH, L), jnp.float32),
        pltpu.VMEM((CH, L), jnp.float32),
        pltpu.SemaphoreType.DMA,
        pltpu.SemaphoreType.DMA,
        pltpu.SemaphoreType.DMA,
        pltpu.SemaphoreType.DMA,
    ],
)
def _hop_kernel(h_hbm, src_hbm, dst_hbm, val_hbm, out_hbm, acc,
                src_v, dst_v, rows_a, rows_b, val_a, val_b,
                sra, srb, sva, svb):
    cid = lax.axis_index("c")
    sid = lax.axis_index("s")
    wid = cid * NS + sid
    pltpu.sync_copy(src_hbm.at[wid], src_v)
    pltpu.sync_copy(dst_hbm.at[wid], dst_v)

    zero = jnp.zeros((L,), jnp.float32)

    @pl.loop(0, CH)
    def _(e):
        for c in range(D // L):
            rows_a[e, pl.ds(c * L, L)] = zero

    @pl.loop(0, RPS // CH)
    def _(b):
        pltpu.sync_copy(rows_a, acc.at[pl.ds(sid * RPS + b * CH, CH)])

    plsc.subcore_barrier()

    def issue(j, rows, vals, sr, sv):
        pltpu.async_copy(h_hbm.at[src_v.at[j]], rows, sr)
        pltpu.async_copy(val_hbm.at[wid, j], vals, sv)

    issue(0, rows_a, val_a, sra, sva)
    issue(1, rows_b, val_b, srb, svb)

    def process(j, rows, vals, sr, sv):
        pltpu.make_async_copy(h_hbm.at[src_v.at[j]], rows, sr).wait()
        pltpu.make_async_copy(val_hbm.at[wid, j], vals, sv).wait()

        @pl.loop(0, CH)
        def _(e):
            v = vals[e, :]
            for c in range(D // L):
                sl = pl.ds(c * L, L)
                rows[e, sl] = rows[e, sl] * v

        pltpu.sync_copy(rows, acc.at[dst_v.at[j]], add=True)

        @pl.when(j + 2 < NCHUNK)
        def _():
            issue(j + 2, rows, vals, sr, sv)

    @pl.loop(0, NCHUNK, step=2)
    def _(j):
        process(j, rows_a, val_a, sra, sva)
        process(j + 1, rows_b, val_b, srb, svb)

    plsc.subcore_barrier()
    pltpu.sync_copy(acc.at[pl.ds(sid * RPS, RPS)],
                    out_hbm.at[cid, pl.ds(sid * RPS, RPS)])


def _blend(h, p0, p1):
    def body(h_ref, p0_ref, p1_ref, o_ref):
        o_ref[...] = ALPHA * h_ref[...] + (1.0 - ALPHA) * (p0_ref[...] + p1_ref[...])

    blk = N_PAD // 8
    return pl.pallas_call(
        body,
        out_shape=jax.ShapeDtypeStruct((N_PAD, D), jnp.float32),
        grid=(8,),
        in_specs=[pl.BlockSpec((blk, D), lambda i: (i, 0))] * 3,
        out_specs=pl.BlockSpec((blk, D), lambda i: (i, 0)),
    )(h, p0, p1)


def kernel(H, edge_index, edge_weight):
    src = edge_index[0]
    dst = edge_index[1]
    pad = E_PAD - E
    pad_idx = jnp.arange(pad, dtype=jnp.int32)
    src_p = jnp.concatenate([src.astype(jnp.int32), pad_idx % N])
    dst_p = jnp.concatenate([dst.astype(jnp.int32), N + pad_idx % (N_PAD - N)])
    w_p = jnp.concatenate([edge_weight.astype(jnp.float32),
                           jnp.zeros((pad,), jnp.float32)])
    src3 = src_p.reshape(NW, NCHUNK, CH)
    dst3 = dst_p.reshape(NW, NCHUNK, CH)
    w3 = w_p.reshape(NW, NCHUNK, CH)
    h_pad = jnp.pad(H.astype(jnp.float32), ((0, N_PAD - N), (0, 0)))

    pden = _denom_partial_kernel(dst3, w3)
    val = _val_kernel(dst3, w3, pden[0], pden[1])

    hw = h_pad
    for _ in range(HOPS):
        p = _hop_kernel(hw, src3, dst3, val)
        hw = _blend(hw, p[0], p[1])
    return hw[:N].astype(H.dtype)


# speedup vs baseline: 5.8987x; 5.8987x over previous
import functools

import jax
import jax.numpy as jnp
from jax import lax
from jax.experimental import pallas as pl
from jax.experimental.pallas import tpu as pltpu
from jax.experimental.pallas import tpu_sc as plsc

N = 10000
E = 320000
D = 128
HOPS = 3
ALPHA = 0.5

NC = 2
NS = 16
NW = NC * NS
CH = 80
NCHUNK = 128
E_PAD = NW * NCHUNK * CH
N_PAD = 10240
RPS = N_PAD // NS
L = 16

_mesh = plsc.VectorSubcoreMesh(core_axis_name="c", subcore_axis_name="s")

_buf_types = dict(
    src_i=pltpu.VMEM((CH,), jnp.int32),
    dst_i=pltpu.VMEM((CH,), jnp.int32),
    vals=pltpu.VMEM((CH * L,), jnp.float32),
    rows=pltpu.VMEM((CH, D), jnp.float32),
    s_idx=pltpu.SemaphoreType.DMA,
    s_rows=pltpu.SemaphoreType.DMA,
)


@functools.partial(
    pl.kernel,
    out_type=jax.ShapeDtypeStruct((NC, N_PAD, D), jnp.float32),
    mesh=_mesh,
    scratch_types=[
        pltpu.VMEM_SHARED((N_PAD, D), jnp.float32),
        dict(_buf_types),
        dict(_buf_types),
    ],
)
def _hop_kernel(h_hbm, src_hbm, dst_hbm, val_hbm, out_hbm, acc, ba, bb):
    cid = lax.axis_index("c")
    sid = lax.axis_index("s")
    wid = cid * NS + sid

    zero = jnp.zeros((L,), jnp.float32)

    @pl.loop(0, CH)
    def _(e):
        for c in range(D // L):
            ba["rows"][e, pl.ds(c * L, L)] = zero

    @pl.loop(0, RPS // CH)
    def _(b):
        pltpu.sync_copy(ba["rows"], acc.at[pl.ds(sid * RPS + b * CH, CH)])

    plsc.subcore_barrier()

    def issue_idx(j, buf):
        pltpu.async_copy(src_hbm.at[wid, j], buf["src_i"], buf["s_idx"])
        pltpu.async_copy(dst_hbm.at[wid, j], buf["dst_i"], buf["s_idx"])
        pltpu.async_copy(val_hbm.at[wid, j], buf["vals"], buf["s_idx"])

    def wait_idx(j, buf):
        pltpu.make_async_copy(src_hbm.at[wid, j], buf["src_i"], buf["s_idx"]).wait()
        pltpu.make_async_copy(dst_hbm.at[wid, j], buf["dst_i"], buf["s_idx"]).wait()
        pltpu.make_async_copy(val_hbm.at[wid, j], buf["vals"], buf["s_idx"]).wait()

    def issue_gather(buf):
        pltpu.async_copy(h_hbm.at[buf["src_i"]], buf["rows"], buf["s_rows"])

    issue_idx(0, ba)
    wait_idx(0, ba)
    issue_gather(ba)
    issue_idx(1, bb)

    def process(j, cur, nxt):
        @pl.when(j + 1 < NCHUNK)
        def _():
            wait_idx(j + 1, nxt)
            issue_gather(nxt)

        pltpu.make_async_copy(h_hbm.at[cur["src_i"]], cur["rows"],
                              cur["s_rows"]).wait()
        rows = cur["rows"]
        vals = cur["vals"]

        @pl.loop(0, CH)
        def _(e):
            v = vals[pl.ds(e * L, L)]
            for c in range(D // L):
                sl = pl.ds(c * L, L)
                rows[e, sl] = rows[e, sl] * v

        pltpu.sync_copy(rows, acc.at[cur["dst_i"]], add=True)

        @pl.when(j + 2 < NCHUNK)
        def _():
            issue_idx(j + 2, cur)

    @pl.loop(0, NCHUNK, step=2)
    def _(j):
        process(j, ba, bb)
        process(j + 1, bb, ba)

    plsc.subcore_barrier()
    pltpu.sync_copy(acc.at[pl.ds(sid * RPS, RPS)],
                    out_hbm.at[cid, pl.ds(sid * RPS, RPS)])


def _blend(h, p0, p1, d0, d1):
    def body(h_ref, p0_ref, p1_ref, d0_ref, d1_ref, o_ref):
        d = jnp.maximum(d0_ref[...] + d1_ref[...], 1e-12)
        agg = (p0_ref[...] + p1_ref[...]) / d
        o_ref[...] = ALPHA * h_ref[...] + (1.0 - ALPHA) * agg

    blk = N_PAD // 8
    return pl.pallas_call(
        body,
        out_shape=jax.ShapeDtypeStruct((N_PAD, D), jnp.float32),
        grid=(8,),
        in_specs=[pl.BlockSpec((blk, D), lambda i: (i, 0))] * 5,
        out_specs=pl.BlockSpec((blk, D), lambda i: (i, 0)),
    )(h, p0, p1, d0, d1)


def kernel(H, edge_index, edge_weight):
    src = edge_index[0]
    dst = edge_index[1]
    pad = E_PAD - E
    pad_idx = jnp.arange(pad, dtype=jnp.int32)
    src_p = jnp.concatenate([src.astype(jnp.int32), pad_idx % N])
    dst_p = jnp.concatenate([dst.astype(jnp.int32), N + pad_idx % (N_PAD - N)])
    w_p = jnp.concatenate([edge_weight.astype(jnp.float32),
                           jnp.zeros((pad,), jnp.float32)])
    src3 = src_p.reshape(NW, NCHUNK, CH)
    dst3 = dst_p.reshape(NW, NCHUNK, CH)
    w16 = jnp.broadcast_to(
        w_p.reshape(NW, NCHUNK, CH, 1), (NW, NCHUNK, CH, L)
    ).reshape(NW, NCHUNK, CH * L)
    h_pad = jnp.pad(H.astype(jnp.float32), ((0, N_PAD - N), (0, 0)))
    ones = jnp.ones((N_PAD, D), jnp.float32)

    pden = _hop_kernel(ones, src3, dst3, w16)

    hw = h_pad
    for _ in range(HOPS):
        p = _hop_kernel(hw, src3, dst3, w16)
        hw = _blend(hw, p[0], p[1], pden[0], pden[1])
    return hw[:N].astype(H.dtype)


# speedup vs baseline: 6.5220x; 1.1057x over previous
import functools

import jax
import jax.numpy as jnp
from jax import lax
from jax.experimental import pallas as pl
from jax.experimental.pallas import tpu as pltpu
from jax.experimental.pallas import tpu_sc as plsc

N = 10000
E = 320000
D = 128
HOPS = 3
ALPHA = 0.5

NC = 2
NS = 16
NW = NC * NS
CH = 80
NCHUNK = 128
E_PAD = NW * NCHUNK * CH
N_PAD = 10240
RPS = N_PAD // NS
L = 16

_mesh = plsc.VectorSubcoreMesh(core_axis_name="c", subcore_axis_name="s")

_buf_types = dict(
    src_i=pltpu.VMEM((CH,), jnp.int32),
    dst_i=pltpu.VMEM((CH,), jnp.int32),
    vals=pltpu.VMEM((CH * L,), jnp.float32),
    rows=pltpu.VMEM((CH, D), jnp.float32),
    s_idx=pltpu.SemaphoreType.DMA,
    s_rows=pltpu.SemaphoreType.DMA,
)


@functools.partial(
    pl.kernel,
    out_type=jax.ShapeDtypeStruct((NC, N_PAD, D), jnp.float32),
    mesh=_mesh,
    scratch_types=[
        pltpu.VMEM_SHARED((N_PAD, D), jnp.float32),
        dict(_buf_types),
        dict(_buf_types),
    ],
)
def _hop_kernel(h_hbm, src_hbm, dst_hbm, val_hbm, out_hbm, acc, ba, bb):
    cid = lax.axis_index("c")
    sid = lax.axis_index("s")
    wid = cid * NS + sid

    zero = jnp.zeros((L,), jnp.float32)

    @pl.loop(0, CH)
    def _(e):
        for c in range(D // L):
            ba["rows"][e, pl.ds(c * L, L)] = zero

    @pl.loop(0, RPS // CH)
    def _(b):
        pltpu.sync_copy(ba["rows"], acc.at[pl.ds(sid * RPS + b * CH, CH)])

    plsc.subcore_barrier()

    def issue_idx(j, buf):
        pltpu.async_copy(src_hbm.at[wid, j], buf["src_i"], buf["s_idx"])
        pltpu.async_copy(dst_hbm.at[wid, j], buf["dst_i"], buf["s_idx"])
        pltpu.async_copy(val_hbm.at[wid, j], buf["vals"], buf["s_idx"])

    def wait_idx(j, buf):
        pltpu.make_async_copy(src_hbm.at[wid, j], buf["src_i"], buf["s_idx"]).wait()
        pltpu.make_async_copy(dst_hbm.at[wid, j], buf["dst_i"], buf["s_idx"]).wait()
        pltpu.make_async_copy(val_hbm.at[wid, j], buf["vals"], buf["s_idx"]).wait()

    def issue_gather(buf):
        pltpu.async_copy(h_hbm.at[buf["src_i"]], buf["rows"], buf["s_rows"])

    issue_idx(0, ba)
    wait_idx(0, ba)
    issue_gather(ba)
    issue_idx(1, bb)

    def process(j, cur, nxt):
        @pl.when(j + 1 < NCHUNK)
        def _():
            wait_idx(j + 1, nxt)
            issue_gather(nxt)

        pltpu.make_async_copy(h_hbm.at[cur["src_i"]], cur["rows"],
                              cur["s_rows"]).wait()
        rows = cur["rows"]
        vals = cur["vals"]

        @pl.loop(0, CH)
        def _(e):
            v = vals[pl.ds(e * L, L)]
            for c in range(D // L):
                sl = pl.ds(c * L, L)
                rows[e, sl] = rows[e, sl] * v

        pltpu.sync_copy(rows, acc.at[cur["dst_i"]], add=True)

        @pl.when(j + 2 < NCHUNK)
        def _():
            issue_idx(j + 2, cur)

    @pl.loop(0, NCHUNK, step=2)
    def _(j):
        process(j, ba, bb)
        process(j + 1, bb, ba)

    plsc.subcore_barrier()
    pltpu.sync_copy(acc.at[pl.ds(sid * RPS, RPS)],
                    out_hbm.at[cid, pl.ds(sid * RPS, RPS)])


@functools.partial(
    pl.kernel,
    out_type=jax.ShapeDtypeStruct((NC, N_PAD, D), jnp.float32),
    mesh=_mesh,
    scratch_types=[
        pltpu.VMEM_SHARED((N_PAD, D), jnp.float32),
        dict(
            dst_i=pltpu.VMEM((CH,), jnp.int32),
            vals=pltpu.VMEM((CH * L,), jnp.float32),
            w2d=pltpu.VMEM((CH, D), jnp.float32),
            s_idx=pltpu.SemaphoreType.DMA,
        ),
        dict(
            dst_i=pltpu.VMEM((CH,), jnp.int32),
            vals=pltpu.VMEM((CH * L,), jnp.float32),
            w2d=pltpu.VMEM((CH, D), jnp.float32),
            s_idx=pltpu.SemaphoreType.DMA,
        ),
    ],
)
def _denom_kernel(dst_hbm, val_hbm, out_hbm, acc, ba, bb):
    cid = lax.axis_index("c")
    sid = lax.axis_index("s")
    wid = cid * NS + sid

    zero = jnp.zeros((L,), jnp.float32)

    @pl.loop(0, CH)
    def _(e):
        for c in range(D // L):
            ba["w2d"][e, pl.ds(c * L, L)] = zero

    @pl.loop(0, RPS // CH)
    def _(b):
        pltpu.sync_copy(ba["w2d"], acc.at[pl.ds(sid * RPS + b * CH, CH)])

    plsc.subcore_barrier()

    def issue_idx(j, buf):
        pltpu.async_copy(dst_hbm.at[wid, j], buf["dst_i"], buf["s_idx"])
        pltpu.async_copy(val_hbm.at[wid, j], buf["vals"], buf["s_idx"])

    def wait_idx(j, buf):
        pltpu.make_async_copy(dst_hbm.at[wid, j], buf["dst_i"], buf["s_idx"]).wait()
        pltpu.make_async_copy(val_hbm.at[wid, j], buf["vals"], buf["s_idx"]).wait()

    issue_idx(0, ba)
    issue_idx(1, bb)

    def process(j, cur, nxt):
        wait_idx(j, cur)

        @pl.loop(0, CH)
        def _(e):
            v = cur["vals"][pl.ds(e * L, L)]
            for c in range(D // L):
                cur["w2d"][e, pl.ds(c * L, L)] = v

        pltpu.sync_copy(cur["w2d"], acc.at[cur["dst_i"]], add=True)

        @pl.when(j + 2 < NCHUNK)
        def _():
            issue_idx(j + 2, cur)

    @pl.loop(0, NCHUNK, step=2)
    def _(j):
        process(j, ba, bb)
        process(j + 1, bb, ba)

    plsc.subcore_barrier()
    pltpu.sync_copy(acc.at[pl.ds(sid * RPS, RPS)],
                    out_hbm.at[cid, pl.ds(sid * RPS, RPS)])


def _blend(h, p0, p1, d0, d1):
    def body(h_ref, p0_ref, p1_ref, d0_ref, d1_ref, o_ref):
        d = jnp.maximum(d0_ref[...] + d1_ref[...], 1e-12)
        agg = (p0_ref[...] + p1_ref[...]) / d
        o_ref[...] = ALPHA * h_ref[...] + (1.0 - ALPHA) * agg

    blk = N_PAD // 8
    return pl.pallas_call(
        body,
        out_shape=jax.ShapeDtypeStruct((N_PAD, D), jnp.float32),
        grid=(8,),
        in_specs=[pl.BlockSpec((blk, D), lambda i: (i, 0))] * 5,
        out_specs=pl.BlockSpec((blk, D), lambda i: (i, 0)),
    )(h, p0, p1, d0, d1)


def kernel(H, edge_index, edge_weight):
    src = edge_index[0]
    dst = edge_index[1]
    pad = E_PAD - E
    pad_idx = jnp.arange(pad, dtype=jnp.int32)
    src_p = jnp.concatenate([src.astype(jnp.int32), pad_idx % N])
    dst_p = jnp.concatenate([dst.astype(jnp.int32), N + pad_idx % (N_PAD - N)])
    w_p = jnp.concatenate([edge_weight.astype(jnp.float32),
                           jnp.zeros((pad,), jnp.float32)])
    src3 = src_p.reshape(NW, NCHUNK, CH)
    dst3 = dst_p.reshape(NW, NCHUNK, CH)
    w16 = jnp.broadcast_to(
        w_p.reshape(NW, NCHUNK, CH, 1), (NW, NCHUNK, CH, L)
    ).reshape(NW, NCHUNK, CH * L)
    h_pad = jnp.pad(H.astype(jnp.float32), ((0, N_PAD - N), (0, 0)))

    pden = _denom_kernel(dst3, w16)

    hw = h_pad
    for _ in range(HOPS):
        p = _hop_kernel(hw, src3, dst3, w16)
        hw = _blend(hw, p[0], p[1], pden[0], pden[1])
    return hw[:N].astype(H.dtype)
